# baseline (speedup 1.0000x reference)
import jax
import jax.numpy as jnp
from jax.experimental import pallas as pl
from jax.experimental.pallas import tpu as pltpu

_BLOCK_ROWS = 64


def _body(w_ref, x_ref, logits_ref, tgt_ref, out_ref):
    i = pl.program_id(0)

    @pl.when(i == 0)
    def _margin():
        lg = logits_ref[...]
        t = tgt_ref[...]
        col = jax.lax.broadcasted_iota(jnp.int32, lg.shape, 1)
        onehot = col == t
        masked = jnp.where(onehot, -jnp.inf, lg)
        max_other = jnp.max(masked, axis=1)
        true_score = jnp.sum(jnp.where(onehot, lg, 0.0), axis=1)
        margin = jnp.maximum(true_score - max_other, -10.0)
        out_ref[0, 0] = 0.0
        out_ref[0, 1] = jnp.sum(margin)

    wt = 127.5 * (jnp.tanh(w_ref[...]) + 1.0)
    d = wt - x_ref[...]
    out_ref[0, 0] += jnp.sum(d * d)


def kernel(w, x, logits, targets):
    n_rows = w.shape[0] * w.shape[1]
    n_cols = w.shape[2] * w.shape[3]
    batch, n_classes = logits.shape
    c_pad = ((n_classes + 127) // 128) * 128
    grid = n_rows // _BLOCK_ROWS

    wf = w.reshape(n_rows, n_cols)
    xf = x.reshape(n_rows, n_cols)
    lg = jnp.pad(logits, ((0, 0), (0, c_pad - n_classes)),
                 constant_values=-jnp.inf)

    out = pl.pallas_call(
        _body,
        grid=(grid,),
        in_specs=[
            pl.BlockSpec((_BLOCK_ROWS, n_cols), lambda i: (i, 0)),
            pl.BlockSpec((_BLOCK_ROWS, n_cols), lambda i: (i, 0)),
            pl.BlockSpec((batch, c_pad), lambda i: (0, 0)),
            pl.BlockSpec((batch, 1), lambda i: (0, 0)),
        ],
        out_specs=pl.BlockSpec(memory_space=pltpu.SMEM),
        out_shape=jax.ShapeDtypeStruct((1, 2), jnp.float32),
        compiler_params=pltpu.CompilerParams(
            dimension_semantics=("arbitrary",),
        ),
    )(wf, xf, lg, targets)

    n_total = n_rows * n_cols
    return out[0, 0] / n_total + 0.5 * out[0, 1] / batch


# speedup vs baseline: 1.4949x; 1.4949x over previous
import jax
import jax.numpy as jnp
from jax.experimental import pallas as pl
from jax.experimental.pallas import tpu as pltpu

_BLOCK_B = 16


def _body(w_ref, x_ref, logits_ref, tgt_ref, out_ref):
    i = pl.program_id(0)

    @pl.when(i == 0)
    def _margin():
        lg = logits_ref[...]
        t = tgt_ref[...]
        col = jax.lax.broadcasted_iota(jnp.int32, lg.shape, 1)
        onehot = col == t
        masked = jnp.where(onehot, -jnp.inf, lg)
        max_other = jnp.max(masked, axis=1)
        true_score = jnp.sum(jnp.where(onehot, lg, 0.0), axis=1)
        margin = jnp.maximum(true_score - max_other, -10.0)
        out_ref[0, 0] = 0.0
        out_ref[0, 1] = jnp.sum(margin)

    wt = 127.5 * (jnp.tanh(w_ref[...]) + 1.0)
    d = wt - x_ref[...]
    out_ref[0, 0] += jnp.sum(d * d)


def kernel(w, x, logits, targets):
    b, ch, h, wd = w.shape
    batch, n_classes = logits.shape
    c_pad = ((n_classes + 127) // 128) * 128
    grid = b // _BLOCK_B

    lg = jnp.pad(logits, ((0, 0), (0, c_pad - n_classes)),
                 constant_values=-jnp.inf)

    out = pl.pallas_call(
        _body,
        grid=(grid,),
        in_specs=[
            pl.BlockSpec((_BLOCK_B, ch, h, wd), lambda i: (i, 0, 0, 0)),
            pl.BlockSpec((_BLOCK_B, ch, h, wd), lambda i: (i, 0, 0, 0)),
            pl.BlockSpec((batch, c_pad), lambda i: (0, 0)),
            pl.BlockSpec((batch, 1), lambda i: (0, 0)),
        ],
        out_specs=pl.BlockSpec(memory_space=pltpu.SMEM),
        out_shape=jax.ShapeDtypeStruct((1, 2), jnp.float32),
        compiler_params=pltpu.CompilerParams(
            dimension_semantics=("arbitrary",),
        ),
    )(w, x, lg, targets)

    n_total = b * ch * h * wd
    return out[0, 0] / n_total + 0.5 * out[0, 1] / batch


# speedup vs baseline: 1.4998x; 1.0033x over previous
import jax
import jax.numpy as jnp
from jax.experimental import pallas as pl
from jax.experimental.pallas import tpu as pltpu

_BLOCK_B = 8


def _body(w_ref, x_ref, logits_ref, tgt_ref, out_ref, acc_ref):
    i = pl.program_id(0)

    @pl.when(i == 0)
    def _margin():
        lg = logits_ref[...]
        t = tgt_ref[...]
        col = jax.lax.broadcasted_iota(jnp.int32, lg.shape, 1)
        onehot = col == t
        masked = jnp.where(onehot, -jnp.inf, lg)
        max_other = jnp.max(masked, axis=1)
        true_score = jnp.sum(jnp.where(onehot, lg, 0.0), axis=1)
        margin = jnp.maximum(true_score - max_other, -10.0)
        out_ref[0, 1] = jnp.sum(margin)
        acc_ref[...] = jnp.zeros_like(acc_ref)

    wt = 127.5 * (jnp.tanh(w_ref[...]) + 1.0)
    d = wt - x_ref[...]
    acc_ref[...] += jnp.sum(d * d, axis=(0, 1))

    @pl.when(i == pl.num_programs(0) - 1)
    def _finish():
        out_ref[0, 0] = jnp.sum(acc_ref[...])


def kernel(w, x, logits, targets):
    b, ch, h, wd = w.shape
    batch, n_classes = logits.shape
    c_pad = ((n_classes + 127) // 128) * 128
    grid = b // _BLOCK_B

    lg = jnp.pad(logits, ((0, 0), (0, c_pad - n_classes)),
                 constant_values=-jnp.inf)

    out = pl.pallas_call(
        _body,
        grid=(grid,),
        in_specs=[
            pl.BlockSpec((_BLOCK_B, ch, h, wd), lambda i: (i, 0, 0, 0)),
            pl.BlockSpec((_BLOCK_B, ch, h, wd), lambda i: (i, 0, 0, 0)),
            pl.BlockSpec((batch, c_pad), lambda i: (0, 0)),
            pl.BlockSpec((batch, 1), lambda i: (0, 0)),
        ],
        out_specs=pl.BlockSpec(memory_space=pltpu.SMEM),
        out_shape=jax.ShapeDtypeStruct((1, 2), jnp.float32),
        scratch_shapes=[pltpu.VMEM((h, wd), jnp.float32)],
        compiler_params=pltpu.CompilerParams(
            dimension_semantics=("arbitrary",),
        ),
    )(w, x, lg, targets)

    n_total = b * ch * h * wd
    return out[0, 0] / n_total + 0.5 * out[0, 1] / batch
